# Initial kernel scaffold; baseline (speedup 1.0000x reference)
import jax
import jax.numpy as jnp
from jax.experimental import pallas as pl


def kernel(input_embeddings, pos_table, gamma, beta):
    raise NotImplementedError("write your pallas kernel here")


# speedup vs baseline: 2.9899x; 2.9899x over previous
import jax
import jax.numpy as jnp
from jax.experimental import pallas as pl


def _body(x_ref, pos_ref, g_ref, b_ref, o_ref):
    x = x_ref[...]
    p = pos_ref[...]
    y = x + p[None, :, :]
    d = y.shape[-1]
    mu = jnp.mean(y, axis=-1, keepdims=True)
    var = jnp.mean(y * y, axis=-1, keepdims=True) - mu * mu
    xhat = (y - mu) * jax.lax.rsqrt(var + 1e-12)
    o_ref[...] = xhat * g_ref[...] + b_ref[...]


def kernel(input_embeddings, pos_table, gamma, beta):
    B, S, D = input_embeddings.shape
    S_BLK = 512
    grid = (S // S_BLK,)
    g2 = gamma.reshape(1, 1, D)
    b2 = beta.reshape(1, 1, D)
    return pl.pallas_call(
        _body,
        grid=grid,
        in_specs=[
            pl.BlockSpec((B, S_BLK, D), lambda i: (0, i, 0)),
            pl.BlockSpec((S_BLK, D), lambda i: (i, 0)),
            pl.BlockSpec((1, 1, D), lambda i: (0, 0, 0)),
            pl.BlockSpec((1, 1, D), lambda i: (0, 0, 0)),
        ],
        out_specs=pl.BlockSpec((B, S_BLK, D), lambda i: (0, i, 0)),
        out_shape=jax.ShapeDtypeStruct((B, S, D), jnp.float32),
    )(input_embeddings, pos_table, g2, b2)
